# baseline (speedup 1.0000x reference)
import functools

import jax
import jax.numpy as jnp
from jax import lax
from jax.experimental import pallas as pl
from jax.experimental.pallas import tpu as pltpu
from jax.experimental.pallas import tpu_sc as plsc

BATCH = 4096
HIST = 200
FEATURES = 64
CHUNK = 128

_NC, _NS = 2, 16
_NW = _NC * _NS

_B = BATCH * HIST
_NCHUNKS = _B // CHUNK
_CH_PER_W = _NCHUNKS // _NW


def _gather_body(table_hbm, idx_hbm, out_hbm, idx_v, rows_v, gsem):
    wid = lax.axis_index("s") * _NC + lax.axis_index("c")
    base = wid * _CH_PER_W
    pltpu.sync_copy(idx_hbm.at[pl.ds(base, _CH_PER_W)], idx_v)

    def step(j, carry):
        pltpu.async_copy(table_hbm.at[idx_v.at[j]], rows_v, gsem).wait()
        pltpu.sync_copy(rows_v, out_hbm.at[pl.ds((base + j) * CHUNK, CHUNK)])
        return carry

    lax.fori_loop(0, _CH_PER_W, step, 0)


_sc_gather = functools.partial(
    pl.kernel,
    out_type=jax.ShapeDtypeStruct((_B, FEATURES), jnp.float32),
    mesh=plsc.VectorSubcoreMesh(core_axis_name="c", subcore_axis_name="s"),
    scratch_types=[
        pltpu.VMEM((_CH_PER_W, CHUNK), jnp.int32),
        pltpu.VMEM((CHUNK, FEATURES), jnp.float32),
        pltpu.SemaphoreType.DMA,
    ],
    compiler_params=pltpu.CompilerParams(use_tc_tiling_on_sc=False),
)(_gather_body)


def kernel(inputs, embedding):
    idx = jnp.asarray(inputs, jnp.int32).reshape(_NCHUNKS, CHUNK)
    out = _sc_gather(embedding, idx)
    return out.reshape(BATCH, HIST, FEATURES)


# speedup vs baseline: 1.1155x; 1.1155x over previous
import functools

import jax
import jax.numpy as jnp
from jax import lax
from jax.experimental import pallas as pl
from jax.experimental.pallas import tpu as pltpu
from jax.experimental.pallas import tpu_sc as plsc

BATCH = 4096
HIST = 200
FEATURES = 64
CHUNK = 128
GBUF = 4
ROWS_BUF = GBUF * CHUNK

_NC, _NS = 2, 16
_NW = _NC * _NS

_B = BATCH * HIST
_NCHUNKS = _B // CHUNK
_CH_PER_W = _NCHUNKS // _NW
_NGROUPS = _CH_PER_W // GBUF
_NITER = _NGROUPS // 2


def _gather_body(table_hbm, idx_hbm, out_hbm, idx_v, buf_a, buf_b, gsem,
                 ssem_a, ssem_b):
    wid = lax.axis_index("s") * _NC + lax.axis_index("c")
    base = wid * _CH_PER_W
    pltpu.sync_copy(idx_hbm.at[pl.ds(base, _CH_PER_W)], idx_v)

    def out_slice(g):
        return out_hbm.at[pl.ds((base + g * GBUF) * CHUNK, ROWS_BUF)]

    def do_group(h, g, buf, ssem):
        @pl.when(h > 0)
        def _():
            pltpu.make_async_copy(buf, out_slice(g - 2), ssem).wait()
        handles = []
        for b in range(GBUF):
            handles.append(pltpu.async_copy(
                table_hbm.at[idx_v.at[g * GBUF + b]],
                buf.at[pl.ds(b * CHUNK, CHUNK)], gsem))
        for handle in handles:
            handle.wait()
        pltpu.async_copy(buf, out_slice(g), ssem)

    def body(h, carry):
        do_group(h, 2 * h, buf_a, ssem_a)
        do_group(h, 2 * h + 1, buf_b, ssem_b)
        return carry

    lax.fori_loop(0, _NITER, body, 0)
    pltpu.make_async_copy(buf_a, out_slice(_NGROUPS - 2), ssem_a).wait()
    pltpu.make_async_copy(buf_b, out_slice(_NGROUPS - 1), ssem_b).wait()


_sc_gather = functools.partial(
    pl.kernel,
    out_type=jax.ShapeDtypeStruct((_B, FEATURES), jnp.float32),
    mesh=plsc.VectorSubcoreMesh(core_axis_name="c", subcore_axis_name="s"),
    scratch_types=[
        pltpu.VMEM((_CH_PER_W, CHUNK), jnp.int32),
        pltpu.VMEM((ROWS_BUF, FEATURES), jnp.float32),
        pltpu.VMEM((ROWS_BUF, FEATURES), jnp.float32),
        pltpu.SemaphoreType.DMA,
        pltpu.SemaphoreType.DMA,
        pltpu.SemaphoreType.DMA,
    ],
    compiler_params=pltpu.CompilerParams(use_tc_tiling_on_sc=False),
)(_gather_body)


def kernel(inputs, embedding):
    idx = jnp.asarray(inputs, jnp.int32).reshape(_NCHUNKS, CHUNK)
    out = _sc_gather(embedding, idx)
    return out.reshape(BATCH, HIST, FEATURES)
